# baseline (speedup 1.0000x reference)
import jax
import jax.numpy as jnp
from jax import lax
from jax.experimental import pallas as pl
from jax.experimental.pallas import tpu as pltpu
from jax.experimental.pallas import tpu_sc as plsc

B = 16384
VOCAB = 1000000
L = 200
HID = 64
C0 = 128
C1 = L - C0
NC = 2
NS = 16
NW = NC * NS
BAGS = B // NW
LANES = 16


def _embsum_body(x_hbm, table_hbm, out_hbm,
                 idx0, idx1, rows0, rows1, outv,
                 semi0, semi1, semg0, semg1):
    wid = lax.axis_index("s") * NC + lax.axis_index("c")
    base = wid * BAGS

    def idx_start(g, idx_ref, sem):
        pltpu.async_copy(x_hbm.at[base + g], idx_ref, sem)

    def idx_wait(idx_ref, sem):
        pltpu.make_async_copy(x_hbm.at[0], idx_ref, sem).wait()

    def gather_start(idx_ref, rows_ref, sem):
        pltpu.async_copy(table_hbm.at[idx_ref.at[pl.ds(0, C0)]],
                         rows_ref.at[pl.ds(0, C0)], sem)
        pltpu.async_copy(table_hbm.at[idx_ref.at[pl.ds(C0, C1)]],
                         rows_ref.at[pl.ds(C0, C1)], sem)

    def gather_wait(rows_ref, sem):
        pltpu.make_async_copy(table_hbm.at[pl.ds(0, L)], rows_ref, sem).wait()

    def accumulate(rows_ref, g):
        def body(i, acc):
            a0, a1, a2, a3 = acc
            for u in range(4):
                r = i * 4 + u
                a0 = a0 + rows_ref[r, pl.ds(0, LANES)]
                a1 = a1 + rows_ref[r, pl.ds(LANES, LANES)]
                a2 = a2 + rows_ref[r, pl.ds(2 * LANES, LANES)]
                a3 = a3 + rows_ref[r, pl.ds(3 * LANES, LANES)]
            return (a0, a1, a2, a3)

        z = jnp.zeros((LANES,), jnp.float32)
        a0, a1, a2, a3 = lax.fori_loop(0, L // 4, body, (z, z, z, z))
        outv[g, pl.ds(0, LANES)] = a0
        outv[g, pl.ds(LANES, LANES)] = a1
        outv[g, pl.ds(2 * LANES, LANES)] = a2
        outv[g, pl.ds(3 * LANES, LANES)] = a3

    idx_start(0, idx0, semi0)
    idx_wait(idx0, semi0)
    gather_start(idx0, rows0, semg0)
    idx_start(1, idx1, semi1)

    def pair(i, carry):
        g0 = i * 2
        gather_wait(rows0, semg0)
        idx_wait(idx1, semi1)
        gather_start(idx1, rows1, semg1)

        @pl.when(g0 + 2 < BAGS)
        def _():
            idx_start(g0 + 2, idx0, semi0)

        accumulate(rows0, g0)

        gather_wait(rows1, semg1)

        @pl.when(g0 + 2 < BAGS)
        def _():
            idx_wait(idx0, semi0)
            gather_start(idx0, rows0, semg0)

        @pl.when(g0 + 3 < BAGS)
        def _():
            idx_start(g0 + 3, idx1, semi1)

        accumulate(rows1, g0 + 1)
        return carry

    lax.fori_loop(0, BAGS // 2, pair, 0)
    pltpu.sync_copy(outv, out_hbm.at[pl.ds(base, BAGS)])


_embsum = pl.kernel(
    _embsum_body,
    out_type=jax.ShapeDtypeStruct((B, HID), jnp.float32),
    mesh=plsc.VectorSubcoreMesh(core_axis_name="c", subcore_axis_name="s"),
    compiler_params=pltpu.CompilerParams(use_tc_tiling_on_sc=False),
    scratch_types=[
        pltpu.VMEM((L,), jnp.int32),
        pltpu.VMEM((L,), jnp.int32),
        pltpu.VMEM((L, HID), jnp.float32),
        pltpu.VMEM((L, HID), jnp.float32),
        pltpu.VMEM((BAGS, HID), jnp.float32),
        pltpu.SemaphoreType.DMA,
        pltpu.SemaphoreType.DMA,
        pltpu.SemaphoreType.DMA,
        pltpu.SemaphoreType.DMA,
    ],
)


def _mm_body(bag_ref, wt_ref, b_ref, o_ref):
    o_ref[...] = jnp.dot(bag_ref[...], wt_ref[...],
                         preferred_element_type=jnp.float32) + b_ref[...]


_MM_BLK = 2048


def _mm(bag, wt, b2):
    return pl.pallas_call(
        _mm_body,
        grid=(B // _MM_BLK,),
        in_specs=[
            pl.BlockSpec((_MM_BLK, HID), lambda i: (i, 0)),
            pl.BlockSpec((HID, HID), lambda i: (0, 0)),
            pl.BlockSpec((1, HID), lambda i: (0, 0)),
        ],
        out_specs=pl.BlockSpec((_MM_BLK, HID), lambda i: (i, 0)),
        out_shape=jax.ShapeDtypeStruct((B, HID), jnp.float32),
    )(bag, wt, b2)


_DT_BLK = 16384


def _detile_body(t_ref, o_ref):
    v = t_ref[...].reshape(_DT_BLK // 2, 2, HID)
    o_ref[:, 0:HID] = v[:, 0, :]
    o_ref[:, HID:2 * HID] = v[:, 1, :]


def _detile(table):
    return pl.pallas_call(
        _detile_body,
        grid=(VOCAB // _DT_BLK,),
        in_specs=[pl.BlockSpec((_DT_BLK, HID), lambda i: (i, 0))],
        out_specs=pl.BlockSpec((_DT_BLK // 2, 2 * HID), lambda i: (i, 0)),
        out_shape=jax.ShapeDtypeStruct((VOCAB // 2, 2 * HID), jnp.float32),
    )(table)


def kernel(x, table, W, b):
    table_lin = _detile(table).reshape(VOCAB, HID)
    sums = _embsum(x.astype(jnp.int32), table_lin)
    wt = W.T.astype(jnp.float32) * (1.0 / L)
    return _mm(sums, wt, b.reshape(1, HID))


# speedup vs baseline: 1.4522x; 1.4522x over previous
import jax
import jax.numpy as jnp
from jax import lax
from jax.experimental import pallas as pl
from jax.experimental.pallas import tpu as pltpu
from jax.experimental.pallas import tpu_sc as plsc

B = 16384
L = 200
HID = 64
C0 = 128
C1 = L - C0
NC = 2
NS = 16
NW = NC * NS
BAGS = B // NW
LANES = 16


CHB = 16
NCH = BAGS // CHB
UNR = 8


def _embsum_body(x_hbm, table_hbm, out_hbm,
                 ib0, ib1, rows0, rows1, rows2, rows3, outv,
                 si0, si1, sg0, sg1, sg2, sg3):
    wid = lax.axis_index("s") * NC + lax.axis_index("c")
    base = wid * BAGS
    rows = [rows0, rows1, rows2, rows3]
    sg = [sg0, sg1, sg2, sg3]

    def idx_load(c, ib, sem):
        pltpu.async_copy(x_hbm.at[pl.ds(base + c * CHB, CHB)], ib, sem)

    def idx_wait(ib, sem):
        pltpu.make_async_copy(x_hbm.at[pl.ds(0, CHB)], ib, sem).wait()

    def gather_start(ib, k, rows_ref, sem):
        pltpu.async_copy(table_hbm.at[ib.at[k, pl.ds(0, C0)]],
                         rows_ref.at[pl.ds(0, C0)], sem)
        pltpu.async_copy(table_hbm.at[ib.at[k, pl.ds(C0, C1)]],
                         rows_ref.at[pl.ds(C0, C1)], sem)

    def gather_wait(rows_ref, sem):
        pltpu.make_async_copy(table_hbm.at[pl.ds(0, L)], rows_ref, sem).wait()

    def accumulate(rows_ref, g):
        def body(i, acc):
            a0, a1, a2, a3 = acc
            for u in range(UNR):
                r = i * UNR + u
                a0 = a0 + rows_ref[r, pl.ds(0, LANES)]
                a1 = a1 + rows_ref[r, pl.ds(LANES, LANES)]
                a2 = a2 + rows_ref[r, pl.ds(2 * LANES, LANES)]
                a3 = a3 + rows_ref[r, pl.ds(3 * LANES, LANES)]
            return (a0, a1, a2, a3)

        z = jnp.zeros((LANES,), jnp.float32)
        a0, a1, a2, a3 = lax.fori_loop(0, L // UNR, body, (z, z, z, z))
        outv[g, pl.ds(0, LANES)] = a0
        outv[g, pl.ds(LANES, LANES)] = a1
        outv[g, pl.ds(2 * LANES, LANES)] = a2
        outv[g, pl.ds(3 * LANES, LANES)] = a3

    idx_load(0, ib0, si0)
    idx_load(1, ib1, si1)
    idx_wait(ib0, si0)
    gather_start(ib0, 0, rows0, sg0)
    gather_start(ib0, 1, rows1, sg1)
    gather_start(ib0, 2, rows2, sg2)

    def outer(i, carry):
        G = base + i * 2 * CHB
        gl = i * 2 * CHB
        for k in range(2 * CHB):
            buf = k % 4
            gather_wait(rows[buf], sg[buf])
            kk = k + 3
            if kk < CHB:
                gather_start(ib0, kk, rows[kk % 4], sg[kk % 4])
            elif kk == CHB:
                idx_wait(ib1, si1)
                gather_start(ib1, 0, rows[kk % 4], sg[kk % 4])
            elif kk < 2 * CHB:
                gather_start(ib1, kk - CHB, rows[kk % 4], sg[kk % 4])
            elif kk == 2 * CHB:
                @pl.when(i < NCH // 2 - 1)
                def _():
                    idx_wait(ib0, si0)
                    gather_start(ib0, 0, rows[0], sg[0])
            else:
                @pl.when(i < NCH // 2 - 1)
                def _(kk=kk):
                    gather_start(ib0, kk - 2 * CHB, rows[kk % 4], sg[kk % 4])
            if k == CHB - 1:
                @pl.when(i < NCH // 2 - 1)
                def _():
                    idx_load(2 * i + 2, ib0, si0)
            if k == 2 * CHB - 1:
                @pl.when(i < NCH // 2 - 1)
                def _():
                    idx_load(2 * i + 3, ib1, si1)
            accumulate(rows[buf], gl + k)
        return carry

    lax.fori_loop(0, NCH // 2, outer, 0)
    pltpu.sync_copy(outv, out_hbm.at[pl.ds(base, BAGS)])


_embsum = pl.kernel(
    _embsum_body,
    out_type=jax.ShapeDtypeStruct((B, HID), jnp.float32),
    mesh=plsc.VectorSubcoreMesh(core_axis_name="c", subcore_axis_name="s"),
    compiler_params=pltpu.CompilerParams(use_tc_tiling_on_sc=False),
    scratch_types=[
        pltpu.VMEM((CHB, L), jnp.int32),
        pltpu.VMEM((CHB, L), jnp.int32),
        pltpu.VMEM((L, HID), jnp.float32),
        pltpu.VMEM((L, HID), jnp.float32),
        pltpu.VMEM((L, HID), jnp.float32),
        pltpu.VMEM((L, HID), jnp.float32),
        pltpu.VMEM((BAGS, HID), jnp.float32),
        pltpu.SemaphoreType.DMA,
        pltpu.SemaphoreType.DMA,
        pltpu.SemaphoreType.DMA,
        pltpu.SemaphoreType.DMA,
        pltpu.SemaphoreType.DMA,
        pltpu.SemaphoreType.DMA,
    ],
)


def _mm_body(bag_ref, wt_ref, b_ref, o_ref):
    o_ref[...] = jnp.dot(bag_ref[...], wt_ref[...],
                         preferred_element_type=jnp.float32) + b_ref[...]


_MM_BLK = 2048


def _mm(bag, wt, b2):
    return pl.pallas_call(
        _mm_body,
        grid=(B // _MM_BLK,),
        in_specs=[
            pl.BlockSpec((_MM_BLK, HID), lambda i: (i, 0)),
            pl.BlockSpec((HID, HID), lambda i: (0, 0)),
            pl.BlockSpec((1, HID), lambda i: (0, 0)),
        ],
        out_specs=pl.BlockSpec((_MM_BLK, HID), lambda i: (i, 0)),
        out_shape=jax.ShapeDtypeStruct((B, HID), jnp.float32),
    )(bag, wt, b2)


def kernel(x, table, W, b):
    sums = _embsum(x.astype(jnp.int32), table)
    wt = W.T.astype(jnp.float32) * (1.0 / L)
    return _mm(sums, wt, b.reshape(1, HID))
